# Initial kernel scaffold; baseline (speedup 1.0000x reference)
import jax
import jax.numpy as jnp
from jax.experimental import pallas as pl


def kernel(x, edge_index, edge_weight, W1, b1, W2, b2):
    raise NotImplementedError("write your pallas kernel here")


# speedup vs baseline: 8.9021x; 8.9021x over previous
import jax
import jax.numpy as jnp
from jax import lax
from jax.experimental import pallas as pl
from jax.experimental.pallas import tpu as pltpu
from jax.experimental.pallas import tpu_sc as plsc

_N = 10000
_E = 320000
_DIN = 128
_D1 = 128
_D2 = 64
_D = _D1 + _D2
_DG = 128

_NC, _NS, _L = 2, 16, 16
_NW = _NC * _NS
_NPAD = 10240
_RPT = _NPAD // _NS
_C = 80

_EPD = _E // _NW
_NCHD = _EPD // _C
_EPA = _E // _NS
_NCHA = _EPA // _C

_RB = 400

_mesh = plsc.VectorSubcoreMesh(
    core_axis_name="c", subcore_axis_name="s", num_cores=_NC, num_subcores=_NS
)


def _deg_body(row_hbm, w_hbm, out_hbm, idx_v, w_v, buf_v, deg_sp):
    c = lax.axis_index("c")
    s = lax.axis_index("s")
    wid = c * _NS + s
    for k in range(_RPT // _L):
        buf_v[pl.ds(k * _L, _L)] = jnp.zeros((_L,), jnp.float32)
    off = pl.multiple_of(s * _RPT, 8)
    pltpu.sync_copy(buf_v, deg_sp.at[pl.ds(off, _RPT)])
    plsc.subcore_barrier()

    def step(i, carry):
        base = pl.multiple_of(wid * _EPD + i * _C, 8)
        pltpu.sync_copy(row_hbm.at[pl.ds(base, _C)], idx_v)
        pltpu.sync_copy(w_hbm.at[pl.ds(base, _C)], w_v)
        pltpu.sync_copy(w_v, deg_sp.at[idx_v], add=True)
        return carry

    lax.fori_loop(0, _NCHD, step, 0)
    plsc.subcore_barrier()
    pltpu.sync_copy(deg_sp.at[pl.ds(off, _RPT)], out_hbm.at[c, pl.ds(off, _RPT)])


@jax.jit
def _deg_call(row, w):
    f = pl.kernel(
        _deg_body,
        out_type=jax.ShapeDtypeStruct((_NC, _NPAD), jnp.float32),
        mesh=_mesh,
        scratch_types=[
            pltpu.VMEM((_C,), jnp.int32),
            pltpu.VMEM((_C,), jnp.float32),
            pltpu.VMEM((_RPT,), jnp.float32),
            pltpu.VMEM_SHARED((_NPAD,), jnp.float32),
        ],
    )
    return f(row, w)


def _prep_body(x_ref, w_ref, dp0_ref, dp1_ref, g_ref, dinv_ref):
    deg = 1.0 + dp0_ref[...] + dp1_ref[...]
    dinv = lax.rsqrt(deg)
    h = jnp.dot(x_ref[...], w_ref[...], preferred_element_type=jnp.float32)
    g = h * dinv
    g_ref[0, ...] = g[:, :_D1]
    g_ref[1, ...] = jnp.concatenate(
        [g[:, _D1:], jnp.zeros((_RB, _DG - _D2), jnp.float32)], axis=1
    )
    dinv_ref[...] = dinv


@jax.jit
def _prep_call(x, wcat, dp0, dp1):
    grid = (_N // _RB,)
    return pl.pallas_call(
        _prep_body,
        grid=grid,
        in_specs=[
            pl.BlockSpec((_RB, _DIN), lambda i: (i, 0)),
            pl.BlockSpec((_DIN, _D), lambda i: (0, 0)),
            pl.BlockSpec((_RB, 1), lambda i: (i, 0)),
            pl.BlockSpec((_RB, 1), lambda i: (i, 0)),
        ],
        out_specs=[
            pl.BlockSpec((_NC, _RB, _DG), lambda i: (0, i, 0)),
            pl.BlockSpec((_RB, 1), lambda i: (i, 0)),
        ],
        out_shape=[
            jax.ShapeDtypeStruct((_NC, _N, _DG), jnp.float32),
            jax.ShapeDtypeStruct((_N, 1), jnp.float32),
        ],
    )(x, wcat, dp0, dp1)


def _agg_body(row_hbm, col_hbm, w_hbm, g_hbm, out_hbm,
              ridx_v, cidx_v, w_v, rows_v, acc_sp, sem):
    c = lax.axis_index("c")
    s = lax.axis_index("s")

    def zrow(j, carry):
        for k in range(_DG // _L):
            rows_v[j, pl.ds(k * _L, _L)] = jnp.zeros((_L,), jnp.float32)
        return carry

    lax.fori_loop(0, _C, zrow, 0)
    for t in range(_RPT // _C):
        off = pl.multiple_of(s * _RPT + t * _C, 8)
        pltpu.sync_copy(rows_v, acc_sp.at[pl.ds(off, _C)])
    plsc.subcore_barrier()

    def step(i, carry):
        base = pl.multiple_of(s * _EPA + i * _C, 8)
        pltpu.sync_copy(row_hbm.at[pl.ds(base, _C)], ridx_v)
        pltpu.sync_copy(col_hbm.at[pl.ds(base, _C)], cidx_v)
        pltpu.sync_copy(w_hbm.at[pl.ds(base, _C)], w_v)
        pltpu.async_copy(g_hbm.at[c].at[ridx_v], rows_v, sem).wait()

        def scale(jj, cc):
            wv = w_v[pl.ds(jj * _L, _L)]
            for t in range(_L):
                j = jj * _L + t
                wj = wv[t]
                for k in range(_DG // _L):
                    rows_v[j, pl.ds(k * _L, _L)] = (
                        rows_v[j, pl.ds(k * _L, _L)] * wj
                    )
            return cc

        lax.fori_loop(0, _C // _L, scale, 0)
        pltpu.sync_copy(rows_v, acc_sp.at[cidx_v], add=True)
        return carry

    lax.fori_loop(0, _NCHA, step, 0)
    plsc.subcore_barrier()

    def wb(t, carry):
        r0 = pl.multiple_of(s * _RPT + t * _C, 8)
        pltpu.sync_copy(acc_sp.at[pl.ds(r0, _C)], rows_v)
        pltpu.sync_copy(rows_v, out_hbm.at[c, pl.ds(r0, _C)])
        return carry

    lax.fori_loop(0, _RPT // _C, wb, 0)


@jax.jit
def _agg_call(row, col, w, g):
    f = pl.kernel(
        _agg_body,
        out_type=jax.ShapeDtypeStruct((_NC, _NPAD, _DG), jnp.float32),
        mesh=_mesh,
        scratch_types=[
            pltpu.VMEM((_C,), jnp.int32),
            pltpu.VMEM((_C,), jnp.int32),
            pltpu.VMEM((_C,), jnp.float32),
            pltpu.VMEM((_C, _DG), jnp.float32),
            pltpu.VMEM_SHARED((_NPAD, _DG), jnp.float32),
            pltpu.SemaphoreType.DMA,
        ],
    )
    return f(row, col, w, g)


def _softplus(y):
    return jnp.maximum(y, 0.0) + jnp.log(1.0 + jnp.exp(-jnp.abs(y))) + 1e-10


def _kl_sum(mean, std):
    return jnp.sum(-jnp.log(std) + 0.5 * (std * std + mean * mean) - 0.5,
                   axis=1, keepdims=True)


def _fin_body(pa_ref, pb_ref, g_ref, dinv_ref, b1_ref, b2_ref,
              out2_ref, ixz1_ref, ixz2_ref):
    dinv = dinv_ref[...]
    of1 = (pa_ref[...] + g_ref[0, ...]) * dinv + b1_ref[...]
    m1 = of1[:, : _D1 // 2]
    s1 = _softplus(of1[:, _D1 // 2:])
    ixz1_ref[...] = _kl_sum(m1, s1)
    of2 = ((pb_ref[...] + g_ref[1, ...]) * dinv)[:, :_D2] + b2_ref[...]
    m2 = of2[:, : _D2 // 2]
    s2 = _softplus(of2[:, _D2 // 2:])
    ixz2_ref[...] = _kl_sum(m2, s2)
    out2_ref[...] = of2


@jax.jit
def _fin_call(pa, pb, g, dinv, b1r, b2r):
    grid = (_N // _RB,)
    return pl.pallas_call(
        _fin_body,
        grid=grid,
        in_specs=[
            pl.BlockSpec((_RB, _DG), lambda i: (i, 0)),
            pl.BlockSpec((_RB, _DG), lambda i: (i, 0)),
            pl.BlockSpec((_NC, _RB, _DG), lambda i: (0, i, 0)),
            pl.BlockSpec((_RB, 1), lambda i: (i, 0)),
            pl.BlockSpec((1, _D1), lambda i: (0, 0)),
            pl.BlockSpec((1, _D2), lambda i: (0, 0)),
        ],
        out_specs=[
            pl.BlockSpec((_RB, _D2), lambda i: (i, 0)),
            pl.BlockSpec((_RB, 1), lambda i: (i, 0)),
            pl.BlockSpec((_RB, 1), lambda i: (i, 0)),
        ],
        out_shape=[
            jax.ShapeDtypeStruct((_N, _D2), jnp.float32),
            jax.ShapeDtypeStruct((_N, 1), jnp.float32),
            jax.ShapeDtypeStruct((_N, 1), jnp.float32),
        ],
    )(pa, pb, g, dinv, b1r, b2r)


def kernel(x, edge_index, edge_weight, W1, b1, W2, b2):
    row = edge_index[0]
    col = edge_index[1]
    wcat = jnp.concatenate([W1, W2], axis=1)
    b1r = b1.reshape(1, _D1)
    b2r = b2.reshape(1, _D2)

    deg_part = _deg_call(row, edge_weight)
    dp0 = deg_part[0, :_N].reshape(_N, 1)
    dp1 = deg_part[1, :_N].reshape(_N, 1)
    g, dinv = _prep_call(x, wcat, dp0, dp1)
    part = _agg_call(row, col, edge_weight, g)
    out2, ixz1, ixz2 = _fin_call(part[0, :_N], part[1, :_N], g, dinv, b1r, b2r)
    return out2, ixz1.reshape(_N), ixz2.reshape(_N), jnp.zeros(())
